# Initial kernel scaffold; baseline (speedup 1.0000x reference)
import jax
import jax.numpy as jnp
from jax.experimental import pallas as pl


def kernel(x, edge_index, W, b):
    raise NotImplementedError("write your pallas kernel here")


# speedup vs baseline: 21.8950x; 21.8950x over previous
import functools

import jax
import jax.numpy as jnp
from jax import lax
from jax.experimental import pallas as pl
from jax.experimental.pallas import tpu as pltpu
from jax.experimental.pallas import tpu_sc as plsc

N = 10000
E = 160000
T = 12
C = 32
D = T * C
NQ = 4
HQ = D // NQ
NP = 10240
NC = 2
NS = 16
EPW_H = 5008
EP_H = 32 * EPW_H
KE = 128
NCH = 80
EPW_S = KE * NCH
EP_S = NS * EPW_S
JUNK = N + 16

def _deg_body(src_hbm, dst_hbm, dego_hbm, degi_hbm,
              ebuf, hist_o, hist_i, rbuf, sh_o, sh_i):
    cid = lax.axis_index("c")
    sid = lax.axis_index("s")
    w = sid * NC + cid
    zero16 = jnp.zeros((16,), jnp.float32)
    one16 = jnp.ones((16,), jnp.float32)

    def zero_hist(i, _):
        hist_o[pl.ds(i * 16, 16)] = zero16
        hist_i[pl.ds(i * 16, 16)] = zero16
        return 0
    lax.fori_loop(0, NP // 16, zero_hist, 0)

    pltpu.sync_copy(src_hbm.at[pl.ds(w * EPW_H, EPW_H)], ebuf)

    def acc_o(j, _):
        idx = ebuf[pl.ds(j * 16, 16)]
        plsc.addupdate_scatter(hist_o, [idx], one16)
        return 0
    lax.fori_loop(0, EPW_H // 16, acc_o, 0)

    pltpu.sync_copy(dst_hbm.at[pl.ds(w * EPW_H, EPW_H)], ebuf)

    def acc_i(j, _):
        idx = ebuf[pl.ds(j * 16, 16)]
        plsc.addupdate_scatter(hist_i, [idx], one16)
        return 0
    lax.fori_loop(0, EPW_H // 16, acc_i, 0)

    pltpu.sync_copy(hist_o, sh_o.at[sid])
    pltpu.sync_copy(hist_i, sh_i.at[sid])
    plsc.subcore_barrier()

    cols = NP // NS

    def reduce_one(sh, hist, out_hbm):
        pltpu.sync_copy(sh.at[:, pl.ds(sid * cols, cols)], rbuf)

        def red(i, _):
            s = rbuf[0, pl.ds(i * 16, 16)]
            for k in range(1, NS):
                s = s + rbuf[k, pl.ds(i * 16, 16)]
            hist[pl.ds(i * 16, 16)] = s
            return 0
        lax.fori_loop(0, cols // 16, red, 0)
        pltpu.sync_copy(hist.at[pl.ds(0, cols)],
                        out_hbm.at[pl.ds(cid * NP + sid * cols, cols)])

    reduce_one(sh_o, hist_o, dego_hbm)
    reduce_one(sh_i, hist_i, degi_hbm)


@functools.lru_cache(maxsize=None)
def _sc_kernels():
    mesh = plsc.VectorSubcoreMesh(
        core_axis_name="c", subcore_axis_name="s",
        num_cores=NC, num_subcores=NS)
    params = pltpu.CompilerParams(needs_layout_passes=False,
                                  use_tc_tiling_on_sc=False)
    deg_kernel = functools.partial(
        pl.kernel,
        out_type=(jax.ShapeDtypeStruct((NC * NP,), jnp.float32),
                  jax.ShapeDtypeStruct((NC * NP,), jnp.float32)),
        mesh=mesh,
        compiler_params=params,
        scratch_types=[
            pltpu.VMEM((EPW_H,), jnp.int32),
            pltpu.VMEM((NP,), jnp.float32),
            pltpu.VMEM((NP,), jnp.float32),
            pltpu.VMEM((NS, NP // NS), jnp.float32),
            pltpu.VMEM_SHARED((NS, NP), jnp.float32),
            pltpu.VMEM_SHARED((NS, NP), jnp.float32),
        ],
    )(_deg_body)
    scat_kernel = functools.partial(
        pl.kernel,
        out_type=jax.ShapeDtypeStruct((NP, D), jnp.float32),
        mesh=mesh,
        compiler_params=params,
        scratch_types=[
            pltpu.VMEM((KE,), jnp.int32),
            pltpu.VMEM((KE,), jnp.int32),
            pltpu.VMEM((KE, HQ), jnp.float32),
            pltpu.SemaphoreType.DMA,
            pltpu.VMEM_SHARED((NP, HQ), jnp.float32),
        ],
    )(_scat_body)
    return deg_kernel, scat_kernel


def _scat_body(y_hbm, src4_hbm, dst_hbm, zeros_hbm, out_hbm,
               idx_v, dst_v, rows_v, sem, acc):
    cid = lax.axis_index("c")
    sid = lax.axis_index("s")
    rows = NP // NS

    for p in range(NQ // NC):
        pltpu.sync_copy(zeros_hbm, rows_v)
        for k in range(rows // KE):
            pltpu.sync_copy(rows_v, acc.at[pl.ds(sid * rows + k * KE, KE)])
        plsc.subcore_barrier()

        q0 = (cid * (NQ // NC) + p) * EP_S

        def body(j, _):
            e0 = sid * EPW_S + j * KE
            pltpu.sync_copy(src4_hbm.at[pl.ds(q0 + e0, KE)], idx_v)
            pltpu.sync_copy(dst_hbm.at[pl.ds(e0, KE)], dst_v)
            pltpu.async_copy(y_hbm.at[idx_v], rows_v, sem).wait()
            pltpu.sync_copy(rows_v, acc.at[dst_v], add=True)
            return 0
        lax.fori_loop(0, NCH, body, 0)
        plsc.subcore_barrier()

        pltpu.sync_copy(
            acc.at[pl.ds(sid * rows, rows)],
            out_hbm.at[pl.ds(sid * rows, rows),
                       pl.ds((cid * (NQ // NC) + p) * HQ, HQ)])
        plsc.subcore_barrier()


def _prep_body(x_ref, d_ref, w_ref, o_ref):
    deg = jnp.sum(d_ref[...], axis=1, keepdims=True)
    norm = lax.rsqrt(jnp.maximum(deg, 1.0))
    o_ref[...] = jnp.dot(x_ref[...] * norm, w_ref[...],
                         preferred_element_type=jnp.float32)


def _epi_body(a_ref, d_ref, b_ref, o_ref):
    deg = jnp.sum(d_ref[...], axis=1, keepdims=True)
    norm = lax.rsqrt(jnp.maximum(deg, 1.0))
    o_ref[...] = jnp.maximum(a_ref[...] * norm + b_ref[...], 0.0)


def kernel(x, edge_index, W, b):
    src = edge_index[0]
    dst = edge_index[1]

    padh = jnp.full((EP_H - E,), JUNK, jnp.int32)
    src_h = jnp.concatenate([src, padh])
    dst_h = jnp.concatenate([dst, padh])
    deg_kernel, scat_kernel = _sc_kernels()
    dego_f, degi_f = deg_kernel(src_h, dst_h)
    dego = dego_f.reshape(NC, NP).T
    degi = degi_f.reshape(NC, NP).T

    src_p = jnp.concatenate([src, jnp.zeros((EP_S - E,), jnp.int32)])
    dst_p = jnp.concatenate([dst, jnp.full((EP_S - E,), JUNK, jnp.int32)])
    base4 = NQ * src_p
    src4 = jnp.concatenate([base4 + q for q in range(NQ)])

    x0 = jnp.pad(x[0], ((0, 0), (0, 0), (0, NP - N)))
    xr = x0.transpose(2, 1, 0).reshape(NP, D)
    W12 = jnp.kron(jnp.eye(T, dtype=W.dtype), W)

    grid = NP // 128
    y = pl.pallas_call(
        _prep_body,
        grid=(grid,),
        in_specs=[pl.BlockSpec((128, D), lambda i: (i, 0)),
                  pl.BlockSpec((128, NC), lambda i: (i, 0)),
                  pl.BlockSpec((D, D), lambda i: (0, 0))],
        out_specs=pl.BlockSpec((128, D), lambda i: (i, 0)),
        out_shape=jax.ShapeDtypeStruct((NP, D), jnp.float32),
    )(xr, dego, W12)

    y_tab = y.reshape(NQ * NP, HQ)
    zeros = jnp.zeros((KE, HQ), jnp.float32)
    agg = scat_kernel(y_tab, src4, dst_p, zeros)

    b12 = jnp.tile(b, T)[None]
    z = pl.pallas_call(
        _epi_body,
        grid=(grid,),
        in_specs=[pl.BlockSpec((128, D), lambda i: (i, 0)),
                  pl.BlockSpec((128, NC), lambda i: (i, 0)),
                  pl.BlockSpec((1, D), lambda i: (0, 0))],
        out_specs=pl.BlockSpec((128, D), lambda i: (i, 0)),
        out_shape=jax.ShapeDtypeStruct((NP, D), jnp.float32),
    )(agg, degi, b12)

    return z[:N].reshape(N, T, C).transpose(2, 1, 0)[None]


# speedup vs baseline: 34.1440x; 1.5594x over previous
import functools

import jax
import jax.numpy as jnp
from jax import lax
from jax.experimental import pallas as pl
from jax.experimental.pallas import tpu as pltpu
from jax.experimental.pallas import tpu_sc as plsc

N = 10000
E = 160000
T = 12
C = 32
D = T * C
NQ = 4
HQ = D // NQ
NP = 10240
NC = 2
NS = 16
EPW_H = 5008
EP_H = 32 * EPW_H
KE = 128
NCH = 80
EPW_S = KE * NCH
EP_S = NS * EPW_S
JUNK = N + 16

def _deg_body(src_hbm, dst_hbm, dego_hbm, degi_hbm,
              ebuf, hist_o, hist_i, rbuf, sh_o, sh_i):
    cid = lax.axis_index("c")
    sid = lax.axis_index("s")
    w = sid * NC + cid
    zero16 = jnp.zeros((16,), jnp.float32)
    one16 = jnp.ones((16,), jnp.float32)

    def zero_hist(i, _):
        hist_o[pl.ds(i * 16, 16)] = zero16
        hist_i[pl.ds(i * 16, 16)] = zero16
        return 0
    lax.fori_loop(0, NP // 16, zero_hist, 0)

    pltpu.sync_copy(src_hbm.at[pl.ds(w * EPW_H, EPW_H)], ebuf)

    def acc_o(j, _):
        idx = ebuf[pl.ds(j * 16, 16)]
        plsc.addupdate_scatter(hist_o, [idx], one16)
        return 0
    lax.fori_loop(0, EPW_H // 16, acc_o, 0)

    pltpu.sync_copy(dst_hbm.at[pl.ds(w * EPW_H, EPW_H)], ebuf)

    def acc_i(j, _):
        idx = ebuf[pl.ds(j * 16, 16)]
        plsc.addupdate_scatter(hist_i, [idx], one16)
        return 0
    lax.fori_loop(0, EPW_H // 16, acc_i, 0)

    pltpu.sync_copy(hist_o, sh_o.at[sid])
    pltpu.sync_copy(hist_i, sh_i.at[sid])
    plsc.subcore_barrier()

    cols = NP // NS

    def reduce_one(sh, hist, out_hbm):
        pltpu.sync_copy(sh.at[:, pl.ds(sid * cols, cols)], rbuf)

        def red(i, _):
            s = rbuf[0, pl.ds(i * 16, 16)]
            for k in range(1, NS):
                s = s + rbuf[k, pl.ds(i * 16, 16)]
            hist[pl.ds(i * 16, 16)] = s
            return 0
        lax.fori_loop(0, cols // 16, red, 0)
        pltpu.sync_copy(hist.at[pl.ds(0, cols)],
                        out_hbm.at[pl.ds(cid * NP + sid * cols, cols)])

    reduce_one(sh_o, hist_o, dego_hbm)
    reduce_one(sh_i, hist_i, degi_hbm)


@functools.lru_cache(maxsize=None)
def _sc_kernels():
    mesh = plsc.VectorSubcoreMesh(
        core_axis_name="c", subcore_axis_name="s",
        num_cores=NC, num_subcores=NS)
    params = pltpu.CompilerParams(needs_layout_passes=False,
                                  use_tc_tiling_on_sc=False)
    deg_kernel = functools.partial(
        pl.kernel,
        out_type=(jax.ShapeDtypeStruct((NC * NP,), jnp.float32),
                  jax.ShapeDtypeStruct((NC * NP,), jnp.float32)),
        mesh=mesh,
        compiler_params=params,
        scratch_types=[
            pltpu.VMEM((EPW_H,), jnp.int32),
            pltpu.VMEM((NP,), jnp.float32),
            pltpu.VMEM((NP,), jnp.float32),
            pltpu.VMEM((NS, NP // NS), jnp.float32),
            pltpu.VMEM_SHARED((NS, NP), jnp.float32),
            pltpu.VMEM_SHARED((NS, NP), jnp.float32),
        ],
    )(_deg_body)
    scat_kernel = functools.partial(
        pl.kernel,
        out_type=jax.ShapeDtypeStruct((NP, D), jnp.float32),
        mesh=mesh,
        compiler_params=params,
        scratch_types=(
            [pltpu.VMEM((NCH, KE), jnp.int32)] * 2
            + [pltpu.VMEM((KE, HQ), jnp.float32)] * NBUF
            + [pltpu.SemaphoreType.DMA] * (2 * NBUF)
            + [pltpu.VMEM_SHARED((NP, HQ), jnp.float32)]),
    )(_scat_body)
    return deg_kernel, scat_kernel


NBUF = 4


def _scat_body(y_hbm, src4_hbm, dst_hbm, zeros_hbm, out_hbm,
               sidx, didx, r0, r1, r2, r3,
               g0, g1, g2, g3, s0, s1, s2, s3, acc):
    cid = lax.axis_index("c")
    sid = lax.axis_index("s")
    rows = NP // NS
    rbufs = (r0, r1, r2, r3)
    gsems = (g0, g1, g2, g3)
    ssems = (s0, s1, s2, s3)

    pltpu.sync_copy(dst_hbm.at[pl.ds(sid * NCH, NCH)], didx)

    for p in range(NQ // NC):
        q = cid * (NQ // NC) + p
        pltpu.sync_copy(src4_hbm.at[pl.ds((q * NS + sid) * NCH, NCH)], sidx)

        pltpu.sync_copy(zeros_hbm, r0)
        for k in range(rows // KE):
            pltpu.sync_copy(r0, acc.at[pl.ds(sid * rows + k * KE, KE)])
        plsc.subcore_barrier()

        for b in range(NBUF):
            pltpu.async_copy(y_hbm.at[sidx.at[b]], rbufs[b], gsems[b])

        def body(jj, _):
            for b in range(NBUF):
                j = jj * NBUF + b
                rb, gs, ss = rbufs[b], gsems[b], ssems[b]
                pltpu.make_async_copy(y_hbm.at[sidx.at[j]], rb, gs).wait()
                pltpu.async_copy(rb, acc.at[didx.at[j]], ss, add=True)

                @pl.when(j < NCH - NBUF)
                def _():
                    pltpu.make_async_copy(rb, acc.at[didx.at[j]], ss).wait()
                    pltpu.async_copy(
                        y_hbm.at[sidx.at[j + NBUF]], rb, gs)
            return 0
        lax.fori_loop(0, NCH // NBUF, body, 0)

        for b in range(NBUF):
            j = NCH - NBUF + b
            pltpu.make_async_copy(rbufs[b], acc.at[didx.at[j]],
                                  ssems[b]).wait()
        plsc.subcore_barrier()

        pltpu.sync_copy(
            acc.at[pl.ds(sid * rows, rows)],
            out_hbm.at[pl.ds(sid * rows, rows), pl.ds(q * HQ, HQ)])
        plsc.subcore_barrier()


def _prep_body(x_ref, d_ref, w_ref, o_ref):
    deg = jnp.sum(d_ref[...], axis=1, keepdims=True)
    norm = lax.rsqrt(jnp.maximum(deg, 1.0))
    o_ref[...] = jnp.dot(x_ref[...] * norm, w_ref[...],
                         preferred_element_type=jnp.float32)


def _epi_body(a_ref, d_ref, b_ref, o_ref):
    deg = jnp.sum(d_ref[...], axis=1, keepdims=True)
    norm = lax.rsqrt(jnp.maximum(deg, 1.0))
    o_ref[...] = jnp.maximum(a_ref[...] * norm + b_ref[...], 0.0)


def kernel(x, edge_index, W, b):
    src = edge_index[0]
    dst = edge_index[1]

    padh = jnp.full((EP_H - E,), JUNK, jnp.int32)
    src_h = jnp.concatenate([src, padh])
    dst_h = jnp.concatenate([dst, padh])
    deg_kernel, scat_kernel = _sc_kernels()
    dego_f, degi_f = deg_kernel(src_h, dst_h)
    dego = dego_f.reshape(NC, NP).T
    degi = degi_f.reshape(NC, NP).T

    src_p = jnp.concatenate([src, jnp.zeros((EP_S - E,), jnp.int32)])
    dst_p = jnp.concatenate([dst, jnp.full((EP_S - E,), JUNK, jnp.int32)])
    base4 = NQ * src_p
    src4 = jnp.concatenate([base4 + q for q in range(NQ)])

    x0 = jnp.pad(x[0], ((0, 0), (0, 0), (0, NP - N)))
    xr = x0.transpose(2, 1, 0).reshape(NP, D)
    W12 = jnp.kron(jnp.eye(T, dtype=W.dtype), W)

    grid = NP // 128
    y = pl.pallas_call(
        _prep_body,
        grid=(grid,),
        in_specs=[pl.BlockSpec((128, D), lambda i: (i, 0)),
                  pl.BlockSpec((128, NC), lambda i: (i, 0)),
                  pl.BlockSpec((D, D), lambda i: (0, 0))],
        out_specs=pl.BlockSpec((128, D), lambda i: (i, 0)),
        out_shape=jax.ShapeDtypeStruct((NP, D), jnp.float32),
    )(xr, dego, W12)

    y_tab = y.reshape(NQ * NP, HQ)
    zeros = jnp.zeros((KE, HQ), jnp.float32)
    agg = scat_kernel(y_tab, src4.reshape(NQ * NS * NCH, KE),
                      dst_p.reshape(NS * NCH, KE), zeros)

    b12 = jnp.tile(b, T)[None]
    z = pl.pallas_call(
        _epi_body,
        grid=(grid,),
        in_specs=[pl.BlockSpec((128, D), lambda i: (i, 0)),
                  pl.BlockSpec((128, NC), lambda i: (i, 0)),
                  pl.BlockSpec((1, D), lambda i: (0, 0))],
        out_specs=pl.BlockSpec((128, D), lambda i: (i, 0)),
        out_shape=jax.ShapeDtypeStruct((NP, D), jnp.float32),
    )(agg, degi, b12)

    return z[:N].reshape(N, T, C).transpose(2, 1, 0)[None]
